# Initial kernel scaffold; baseline (speedup 1.0000x reference)
import jax
import jax.numpy as jnp
from jax.experimental import pallas as pl


def kernel(x, codebooks, ema_cluster_size):
    raise NotImplementedError("write your pallas kernel here")


# speedup vs baseline: 1.2760x; 1.2760x over previous
import jax
import jax.numpy as jnp
from jax.experimental import pallas as pl
from jax.experimental.pallas import tpu as pltpu

_B = 4
_N = 1024
_H = 8
_M = 2048
_D = 32
_BN = _B * _N
_T = 1024
_NT = _BN // _T
_EXPIRE = 2.0


def _vqe_body(q_ref, ct_ref, ema_ref,
              idx_ref, s_ref, loss_ref, perp_ref, repl_ref,
              hist_ref):
    h = pl.program_id(0)
    t = pl.program_id(1)

    q = q_ref[0]
    ct = ct_ref[0]

    dot = jax.lax.dot_general(
        q, ct, (((1,), (0,)), ((), ())),
        preferred_element_type=jnp.float32)
    l2q = jnp.sum(q * q, axis=1, keepdims=True)
    l2c = jnp.sum(ct * ct, axis=0, keepdims=True)
    sim = -(l2q + l2c - 2.0 * dot)

    iota = jax.lax.broadcasted_iota(jnp.int32, sim.shape, 1)
    maxv = jnp.max(sim, axis=1, keepdims=True)
    cand = jnp.where(sim == maxv, iota, _M)
    idx = jnp.min(cand, axis=1, keepdims=True)
    idx_ref[0] = idx

    counts = jnp.sum(jnp.where(idx == iota, 1.0, 0.0),
                     axis=0, keepdims=True)

    s_row = jax.lax.dot_general(
        jnp.ones((1, _M), jnp.float32), ct, (((1,), (1,)), ((), ())),
        preferred_element_type=jnp.float32)

    diff = q - s_row
    partial = jnp.sum(diff * diff).reshape(1, 1, 1)

    @pl.when(jnp.logical_and(h == 0, t == 0))
    def _init_loss():
        loss_ref[...] = jnp.zeros_like(loss_ref)

    loss_ref[...] += partial

    @pl.when(t == 0)
    def _head_start():
        hist_ref[...] = counts
        s_ref[...] = s_row.reshape(1, 1, _D)
        repl_ref[...] = jnp.sum(
            jnp.where(ema_ref[...] < _EXPIRE, 1.0, 0.0)).reshape(1, 1, 1)

    @pl.when(t != 0)
    def _head_accum():
        hist_ref[...] += counts

    @pl.when(t == _NT - 1)
    def _head_finish():
        mean = hist_ref[...] * (1.0 / _BN)
        ent = mean * jnp.log(mean + 1e-10)
        perp_ref[...] = jnp.exp(-jnp.sum(ent)).reshape(1, 1, 1)


def kernel(x, codebooks, ema_cluster_size):
    qh = x.reshape(_B, _N, _H, _D).transpose(2, 0, 1, 3).reshape(_H, _BN, _D)
    ct = codebooks.transpose(0, 2, 1)

    idx, s, loss_sum, perp, repl = pl.pallas_call(
        _vqe_body,
        grid=(_H, _NT),
        in_specs=[
            pl.BlockSpec((1, _T, _D), lambda h, t: (h, t, 0)),
            pl.BlockSpec((1, _D, _M), lambda h, t: (h, 0, 0)),
            pl.BlockSpec((1, 1, _M), lambda h, t: (h, 0, 0)),
        ],
        out_specs=[
            pl.BlockSpec((1, _T, 1), lambda h, t: (h, t, 0)),
            pl.BlockSpec((1, 1, _D), lambda h, t: (h, 0, 0)),
            pl.BlockSpec((1, 1, 1), lambda h, t: (0, 0, 0)),
            pl.BlockSpec((1, 1, 1), lambda h, t: (h, 0, 0)),
            pl.BlockSpec((1, 1, 1), lambda h, t: (h, 0, 0)),
        ],
        out_shape=[
            jax.ShapeDtypeStruct((_H, _BN, 1), jnp.int32),
            jax.ShapeDtypeStruct((_H, 1, _D), jnp.float32),
            jax.ShapeDtypeStruct((1, 1, 1), jnp.float32),
            jax.ShapeDtypeStruct((_H, 1, 1), jnp.float32),
            jax.ShapeDtypeStruct((_H, 1, 1), jnp.float32),
        ],
        scratch_shapes=[pltpu.VMEM((1, _M), jnp.float32)],
    )(qh, ct, ema_cluster_size.reshape(_H, 1, _M))

    codebook_indices = idx.reshape(_H, _B, _N).transpose(1, 0, 2)
    out = jnp.broadcast_to(s.reshape(1, 1, _H * _D), (_B, _N, _H * _D))
    loss = loss_sum.reshape(()) / float(_B * _N * _H * _D)
    perplexity = perp.reshape(_H)
    replaced = repl.reshape(_H).astype(jnp.int32)
    return out, codebook_indices, loss, perplexity, replaced


# speedup vs baseline: 1.4175x; 1.1109x over previous
import jax
import jax.numpy as jnp
from jax.experimental import pallas as pl
from jax.experimental.pallas import tpu as pltpu

_B = 4
_N = 1024
_H = 8
_M = 2048
_D = 32
_BN = _B * _N
_T = 1024
_NT = _BN // _T
_EXPIRE = 2.0


_MHI = 16
_MLO = 128


def _vqe_body(q_ref, ct_ref, ema_ref,
              idx_ref, s_ref, loss_ref, perp_ref, repl_ref,
              hist_ref, srow_ref):
    h = pl.program_id(0)
    t = pl.program_id(1)

    q = q_ref[0]
    ct = ct_ref[0]

    dot = jax.lax.dot_general(
        q, ct, (((1,), (0,)), ((), ())),
        preferred_element_type=jnp.float32)
    l2q = jnp.sum(q * q, axis=1, keepdims=True)
    l2c = jnp.sum(ct * ct, axis=0, keepdims=True)
    sim = -(l2q + l2c - 2.0 * dot)

    iota = jax.lax.broadcasted_iota(jnp.int32, sim.shape, 1)
    maxv = jnp.max(sim, axis=1, keepdims=True)
    cand = jnp.where(sim == maxv, iota, _M)
    idx = jnp.min(cand, axis=1, keepdims=True)
    idx_ref[0] = idx

    hi = jax.lax.shift_right_logical(idx, 7)
    lo = jnp.bitwise_and(idx, _MLO - 1)
    oh_hi = (hi == jax.lax.broadcasted_iota(jnp.int32, (_T, _MHI), 1)
             ).astype(jnp.float32)
    oh_lo = (lo == jax.lax.broadcasted_iota(jnp.int32, (_T, _MLO), 1)
             ).astype(jnp.float32)
    counts = jax.lax.dot_general(
        oh_hi, oh_lo, (((0,), (0,)), ((), ())),
        preferred_element_type=jnp.float32)

    @pl.when(t == 0)
    def _head_start():
        s_row = jax.lax.dot_general(
            jnp.ones((1, _M), jnp.float32), ct, (((1,), (1,)), ((), ())),
            preferred_element_type=jnp.float32)
        srow_ref[...] = s_row
        s_ref[...] = s_row.reshape(1, 1, _D)
        hist_ref[...] = counts
        repl_ref[...] = jnp.sum(
            jnp.where(ema_ref[...] < _EXPIRE, 1.0, 0.0)).reshape(1, 1, 1)

    @pl.when(t != 0)
    def _head_accum():
        hist_ref[...] += counts

    diff = q - srow_ref[...]
    partial = jnp.sum(diff * diff).reshape(1, 1, 1)

    @pl.when(jnp.logical_and(h == 0, t == 0))
    def _init_loss():
        loss_ref[...] = jnp.zeros_like(loss_ref)

    loss_ref[...] += partial

    @pl.when(t == _NT - 1)
    def _head_finish():
        mean = hist_ref[...] * (1.0 / _BN)
        ent = mean * jnp.log(mean + 1e-10)
        perp_ref[...] = jnp.exp(-jnp.sum(ent)).reshape(1, 1, 1)


def kernel(x, codebooks, ema_cluster_size):
    qh = x.reshape(_B, _N, _H, _D).transpose(2, 0, 1, 3).reshape(_H, _BN, _D)
    ct = codebooks.transpose(0, 2, 1)

    idx, s, loss_sum, perp, repl = pl.pallas_call(
        _vqe_body,
        grid=(_H, _NT),
        in_specs=[
            pl.BlockSpec((1, _T, _D), lambda h, t: (h, t, 0)),
            pl.BlockSpec((1, _D, _M), lambda h, t: (h, 0, 0)),
            pl.BlockSpec((1, 1, _M), lambda h, t: (h, 0, 0)),
        ],
        out_specs=[
            pl.BlockSpec((1, _T, 1), lambda h, t: (h, t, 0)),
            pl.BlockSpec((1, 1, _D), lambda h, t: (h, 0, 0)),
            pl.BlockSpec((1, 1, 1), lambda h, t: (0, 0, 0)),
            pl.BlockSpec((1, 1, 1), lambda h, t: (h, 0, 0)),
            pl.BlockSpec((1, 1, 1), lambda h, t: (h, 0, 0)),
        ],
        out_shape=[
            jax.ShapeDtypeStruct((_H, _BN, 1), jnp.int32),
            jax.ShapeDtypeStruct((_H, 1, _D), jnp.float32),
            jax.ShapeDtypeStruct((1, 1, 1), jnp.float32),
            jax.ShapeDtypeStruct((_H, 1, 1), jnp.float32),
            jax.ShapeDtypeStruct((_H, 1, 1), jnp.float32),
        ],
        scratch_shapes=[pltpu.VMEM((_MHI, _MLO), jnp.float32),
                        pltpu.VMEM((1, _D), jnp.float32)],
    )(qh, ct, ema_cluster_size.reshape(_H, 1, _M))

    codebook_indices = idx.reshape(_H, _B, _N).transpose(1, 0, 2)
    out = jnp.broadcast_to(s.reshape(1, 1, _H * _D), (_B, _N, _H * _D))
    loss = loss_sum.reshape(()) / float(_B * _N * _H * _D)
    perplexity = perp.reshape(_H)
    replaced = repl.reshape(_H).astype(jnp.int32)
    return out, codebook_indices, loss, perplexity, replaced


# speedup vs baseline: 1.5312x; 1.0802x over previous
import jax
import jax.numpy as jnp
from jax.experimental import pallas as pl
from jax.experimental.pallas import tpu as pltpu

_B = 4
_N = 1024
_H = 8
_M = 2048
_D = 32
_BN = _B * _N
_T = 1024
_NT = _BN // _T
_EXPIRE = 2.0


_MHI = 16
_MLO = 128


def _vqe_body(q_ref, ct_ref, ema_ref,
              idx_ref, s_ref, loss_ref, perp_ref, repl_ref,
              hist_ref, srow_ref):
    h = pl.program_id(0)
    t = pl.program_id(1)

    q = q_ref[0]
    ct = ct_ref[0]

    dot = jax.lax.dot_general(
        q, ct, (((1,), (0,)), ((), ())),
        preferred_element_type=jnp.float32)
    l2q = jnp.sum(q * q, axis=1, keepdims=True)
    l2c = jnp.sum(ct * ct, axis=0, keepdims=True)
    sim = -(l2q + l2c - 2.0 * dot)

    idx = jnp.argmax(sim, axis=1).reshape(_T, 1)
    idx_ref[0] = idx

    hi = jax.lax.shift_right_logical(idx, 7)
    lo = jnp.bitwise_and(idx, _MLO - 1)
    oh_hi = (hi == jax.lax.broadcasted_iota(jnp.int32, (_T, _MHI), 1)
             ).astype(jnp.float32)
    oh_lo = (lo == jax.lax.broadcasted_iota(jnp.int32, (_T, _MLO), 1)
             ).astype(jnp.float32)
    counts = jax.lax.dot_general(
        oh_hi, oh_lo, (((0,), (0,)), ((), ())),
        preferred_element_type=jnp.float32)

    @pl.when(t == 0)
    def _head_start():
        s_row = jax.lax.dot_general(
            jnp.ones((1, _M), jnp.float32), ct, (((1,), (1,)), ((), ())),
            preferred_element_type=jnp.float32)
        srow_ref[...] = s_row
        s_ref[...] = s_row.reshape(1, 1, _D)
        hist_ref[...] = counts
        repl_ref[...] = jnp.sum(
            jnp.where(ema_ref[...] < _EXPIRE, 1.0, 0.0)).reshape(1, 1, 1)

    @pl.when(t != 0)
    def _head_accum():
        hist_ref[...] += counts

    diff = q - srow_ref[...]
    partial = jnp.sum(diff * diff).reshape(1, 1, 1)

    @pl.when(jnp.logical_and(h == 0, t == 0))
    def _init_loss():
        loss_ref[...] = jnp.zeros_like(loss_ref)

    loss_ref[...] += partial

    @pl.when(t == _NT - 1)
    def _head_finish():
        mean = hist_ref[...] * (1.0 / _BN)
        ent = mean * jnp.log(mean + 1e-10)
        perp_ref[...] = jnp.exp(-jnp.sum(ent)).reshape(1, 1, 1)


def kernel(x, codebooks, ema_cluster_size):
    qh = x.reshape(_B, _N, _H, _D).transpose(2, 0, 1, 3).reshape(_H, _BN, _D)
    ct = codebooks.transpose(0, 2, 1)

    idx, s, loss_sum, perp, repl = pl.pallas_call(
        _vqe_body,
        grid=(_H, _NT),
        in_specs=[
            pl.BlockSpec((1, _T, _D), lambda h, t: (h, t, 0)),
            pl.BlockSpec((1, _D, _M), lambda h, t: (h, 0, 0)),
            pl.BlockSpec((1, 1, _M), lambda h, t: (h, 0, 0)),
        ],
        out_specs=[
            pl.BlockSpec((1, _T, 1), lambda h, t: (h, t, 0)),
            pl.BlockSpec((1, 1, _D), lambda h, t: (h, 0, 0)),
            pl.BlockSpec((1, 1, 1), lambda h, t: (0, 0, 0)),
            pl.BlockSpec((1, 1, 1), lambda h, t: (h, 0, 0)),
            pl.BlockSpec((1, 1, 1), lambda h, t: (h, 0, 0)),
        ],
        out_shape=[
            jax.ShapeDtypeStruct((_H, _BN, 1), jnp.int32),
            jax.ShapeDtypeStruct((_H, 1, _D), jnp.float32),
            jax.ShapeDtypeStruct((1, 1, 1), jnp.float32),
            jax.ShapeDtypeStruct((_H, 1, 1), jnp.float32),
            jax.ShapeDtypeStruct((_H, 1, 1), jnp.float32),
        ],
        scratch_shapes=[pltpu.VMEM((_MHI, _MLO), jnp.float32),
                        pltpu.VMEM((1, _D), jnp.float32)],
    )(qh, ct, ema_cluster_size.reshape(_H, 1, _M))

    codebook_indices = idx.reshape(_H, _B, _N).transpose(1, 0, 2)
    out = jnp.broadcast_to(s.reshape(1, 1, _H * _D), (_B, _N, _H * _D))
    loss = loss_sum.reshape(()) / float(_B * _N * _H * _D)
    perplexity = perp.reshape(_H)
    replaced = repl.reshape(_H).astype(jnp.int32)
    return out, codebook_indices, loss, perplexity, replaced
